# Initial kernel scaffold; baseline (speedup 1.0000x reference)
import jax
import jax.numpy as jnp
from jax.experimental import pallas as pl


def kernel(x, edge_index, batch, W1, b1, W2, b2, Wr, br, Wa1, ba1, Wa2, ba2, Wg, bg):
    raise NotImplementedError("write your pallas kernel here")


# speedup vs baseline: 21.3090x; 21.3090x over previous
import functools

import jax
import jax.numpy as jnp
from jax import lax
from jax.experimental import pallas as pl
from jax.experimental.pallas import tpu as pltpu
from jax.experimental.pallas import tpu_sc as plsc

_N = 10000
_E = 320000
_F = 128
_G = 64
_NC = 2
_NS = 16
_K = 80
_NCH = _E // (_NC * _NS * _K)
_NP = 10112
_RS = _NP // _NS
_ZR = 8
_DW = 16

_f32 = jnp.float32


def _fill_rows(buf, n_rows, n_cols, value):
    vec = jnp.full((16,), value, _f32)

    def body(r, _):
        for cb in range(n_cols // 16):
            buf[r, pl.ds(cb * 16, 16)] = vec
        return 0

    lax.fori_loop(0, n_rows, body, 0)


def _sc_degree(edge_dst):
    mesh = plsc.VectorSubcoreMesh(core_axis_name="c", subcore_axis_name="s")

    @functools.partial(
        pl.kernel,
        out_type=jax.ShapeDtypeStruct((_NC, _NP, _DW), _f32),
        mesh=mesh,
        scratch_types=[
            pltpu.VMEM((_NCH, _K), jnp.int32),
            pltpu.VMEM((_K, _DW), _f32),
            pltpu.VMEM((_ZR, _DW), _f32),
            pltpu.VMEM_SHARED((_NP, _DW), _f32),
        ],
    )
    def deg_kernel(dst_hbm, out_hbm, dst_v, ones_v, zbuf, accum):
        c = lax.axis_index("c")
        s = lax.axis_index("s")
        _fill_rows(zbuf, _ZR, _DW, 0.0)
        _fill_rows(ones_v, _K, _DW, 1.0)

        def zcopy(i, _):
            pltpu.sync_copy(zbuf, accum.at[pl.ds(s * _RS + i * _ZR, _ZR)])
            return 0

        lax.fori_loop(0, _RS // _ZR, zcopy, 0)
        pltpu.sync_copy(dst_hbm.at[c, s], dst_v)
        plsc.subcore_barrier()

        def step(j, _):
            pltpu.sync_copy(ones_v, accum.at[dst_v.at[j]], add=True)
            return 0

        lax.fori_loop(0, _NCH, step, 0)
        plsc.subcore_barrier()
        pltpu.sync_copy(accum.at[pl.ds(s * _RS, _RS)],
                        out_hbm.at[c, pl.ds(s * _RS, _RS)])

    return deg_kernel(edge_dst)


def _sc_aggregate(table, edge_src, edge_dst):
    mesh = plsc.VectorSubcoreMesh(core_axis_name="c", subcore_axis_name="s")

    @functools.partial(
        pl.kernel,
        out_type=jax.ShapeDtypeStruct((_NC, _NP, _F), _f32),
        mesh=mesh,
        scratch_types=[
            pltpu.VMEM((_NCH, _K), jnp.int32),
            pltpu.VMEM((_NCH, _K), jnp.int32),
            pltpu.VMEM((_K, _F), _f32),
            pltpu.VMEM((_ZR, _F), _f32),
            pltpu.VMEM_SHARED((_NP, _F), _f32),
            pltpu.SemaphoreType.DMA,
        ],
    )
    def agg_kernel(table_hbm, src_hbm, dst_hbm, out_hbm,
                   src_v, dst_v, buf, zbuf, accum, sem):
        c = lax.axis_index("c")
        s = lax.axis_index("s")
        _fill_rows(zbuf, _ZR, _F, 0.0)

        def zcopy(i, _):
            pltpu.sync_copy(zbuf, accum.at[pl.ds(s * _RS + i * _ZR, _ZR)])
            return 0

        lax.fori_loop(0, _RS // _ZR, zcopy, 0)
        pltpu.sync_copy(src_hbm.at[c, s], src_v)
        pltpu.sync_copy(dst_hbm.at[c, s], dst_v)
        plsc.subcore_barrier()

        def step(j, _):
            pltpu.async_copy(table_hbm.at[src_v.at[j]], buf, sem).wait()
            pltpu.sync_copy(buf, accum.at[dst_v.at[j]], add=True)
            return 0

        lax.fori_loop(0, _NCH, step, 0)
        plsc.subcore_barrier()
        pltpu.sync_copy(accum.at[pl.ds(s * _RS, _RS)],
                        out_hbm.at[c, pl.ds(s * _RS, _RS)])

    return agg_kernel(table, edge_src, edge_dst)


def _tc_prepare(degp, x):
    def body(degp_ref, x_ref, dinv_ref, xs_ref):
        deg = degp_ref[0, pl.ds(0, _N), :] + degp_ref[1, pl.ds(0, _N), :]
        dinv = lax.rsqrt(deg + 1.0)
        dinv_ref[...] = dinv
        xs_ref[...] = x_ref[...] * dinv[:, 0:1]

    return pl.pallas_call(
        body,
        out_shape=(jax.ShapeDtypeStruct((_N, _DW), _f32),
                   jax.ShapeDtypeStruct((_N, _F), _f32)),
    )(degp, x)


def _tc_conv1(p, xs, dinv, W1, b1):
    def body(p_ref, xs_ref, dinv_ref, W1_ref, b1_ref, h1_ref, h1s_ref):
        d = dinv_ref[:, 0:1]
        agg = (p_ref[0, pl.ds(0, _N), :] + p_ref[1, pl.ds(0, _N), :]
               + xs_ref[...]) * d
        h1 = jnp.dot(agg, W1_ref[...], preferred_element_type=_f32)
        h1 = jnp.maximum(h1 + b1_ref[...], 0.0)
        h1_ref[...] = h1
        h1s_ref[...] = h1 * d

    return pl.pallas_call(
        body,
        out_shape=(jax.ShapeDtypeStruct((_N, _F), _f32),
                   jax.ShapeDtypeStruct((_N, _F), _f32)),
    )(p, xs, dinv, W1, b1)


def _tc_head(p, h1s, h1, dinv, batch2d,
             W2, b2, Wr, br, Wa1, ba1, Wa2, ba2, Wg, bg):
    def body(p_ref, h1s_ref, h1_ref, dinv_ref, b_ref,
             W2_ref, b2_ref, Wr_ref, br_ref, Wa1_ref, ba1_ref,
             Wa2_ref, ba2_ref, Wg_ref, bg_ref, emb_ref, logit_ref):
        d = dinv_ref[:, 0:1]
        agg = (p_ref[0, pl.ds(0, _N), :] + p_ref[1, pl.ds(0, _N), :]
               + h1s_ref[...]) * d
        h2 = jnp.dot(agg, W2_ref[...], preferred_element_type=_f32) + b2_ref[...]
        h2 = h2 + jnp.dot(h1_ref[...], Wr_ref[...],
                          preferred_element_type=_f32) + br_ref[...]
        h2 = jnp.maximum(h2, 0.0)
        a = jnp.maximum(
            jnp.dot(h2, Wa1_ref[...], preferred_element_type=_f32) + ba1_ref[...],
            0.0)
        nw = jnp.dot(a, Wa2_ref[...], preferred_element_type=_f32) + ba2_ref[...]
        w = jax.nn.sigmoid(nw)
        wx = h2 * w
        gids = lax.broadcasted_iota(jnp.int32, (_N, _G), 1)
        oh = (b_ref[...] == gids).astype(_f32)
        dn = (((0,), (0,)), ((), ()))
        sums = lax.dot_general(oh, wx, dn, preferred_element_type=_f32)
        cnt = lax.dot_general(oh, jnp.ones((_N, 1), _f32), dn,
                              preferred_element_type=_f32)
        emb = sums / jnp.maximum(cnt, 1.0)
        emb_ref[...] = emb
        logit_ref[...] = jnp.dot(emb, Wg_ref[...],
                                 preferred_element_type=_f32) + bg_ref[...]

    return pl.pallas_call(
        body,
        out_shape=(jax.ShapeDtypeStruct((_G, 256), _f32),
                   jax.ShapeDtypeStruct((_G, 16), _f32)),
    )(p, h1s, h1, dinv, batch2d,
      W2, b2, Wr, br, Wa1, ba1, Wa2, ba2, Wg, bg)


def kernel(x, edge_index, batch, W1, b1, W2, b2, Wr, br,
           Wa1, ba1, Wa2, ba2, Wg, bg):
    src = edge_index[0].reshape(_NC, _NS, _NCH, _K)
    dst = edge_index[1].reshape(_NC, _NS, _NCH, _K)
    degp = _sc_degree(dst)
    dinv, xs = _tc_prepare(degp, x)
    p1 = _sc_aggregate(xs, src, dst)
    h1, h1s = _tc_conv1(p1, xs, dinv, W1, b1.reshape(1, -1))
    p2 = _sc_aggregate(h1s, src, dst)
    emb, logits = _tc_head(
        p2, h1s, h1, dinv, batch.reshape(-1, 1),
        W2, b2.reshape(1, -1), Wr, br.reshape(1, -1),
        Wa1, ba1.reshape(1, -1), Wa2, ba2.reshape(1, -1),
        Wg, bg.reshape(1, -1))
    return emb, logits
